# Initial kernel scaffold; baseline (speedup 1.0000x reference)
import jax
import jax.numpy as jnp
from jax.experimental import pallas as pl


def kernel(scores, label, seqlen):
    raise NotImplementedError("write your pallas kernel here")


# speedup vs baseline: 2.2242x; 2.2242x over previous
import jax
import jax.numpy as jnp
from jax import lax
from jax.experimental import pallas as pl
from jax.experimental.pallas import tpu as pltpu
from jax.experimental.pallas import tpu_sc as plsc

B, N = 64, 32768
K = 8
L = 16
NC, NS = 2, 16
NW = NC * NS
ROWS_PER_W = B // NW
CHUNKS = N // L
CHAINS = 8
STEPS = CHUNKS // CHAINS

NEG_INF = float("-inf")


def _merge(a, b):
    return jnp.sort(jnp.maximum(a, lax.rev(b, (0,))))


def _sc_topk_body(scores_hbm, seqlen_hbm, label_hbm, out_hbm,
                  row0_v, row1_v, seqlen_v, label_v, res_v, sem0, sem1):
    c = lax.axis_index("c")
    s = lax.axis_index("s")
    wid = s * NC + c
    r0 = wid * ROWS_PER_W

    cp0 = pltpu.async_copy(scores_hbm.at[r0], row0_v, sem0)
    cp1 = pltpu.async_copy(scores_hbm.at[r0 + 1], row1_v, sem1)
    pltpu.sync_copy(seqlen_hbm, seqlen_v)
    pltpu.sync_copy(label_hbm, label_v)

    lane = lax.iota(jnp.int32, L)

    def process_row(rowbuf, r):
        ridx = jnp.full((L,), r, jnp.int32)
        seql = plsc.load_gather(seqlen_v, [ridx])
        lab = plsc.load_gather(label_v, [ridx])

        def step(g, chains):
            base = g * (CHAINS * L)
            new_chains = []
            for j in range(CHAINS):
                off = base + j * L
                x = rowbuf[pl.ds(off, L)]
                idxv = lane + off
                x = jnp.where(idxv < seql, x, NEG_INF)
                new_chains.append(_merge(chains[j], jnp.sort(x)))
            return tuple(new_chains)

        init = tuple(jnp.full((L,), NEG_INF, jnp.float32) for _ in range(CHAINS))
        chains = list(lax.fori_loop(0, STEPS, step, init))
        while len(chains) > 1:
            chains = [_merge(chains[2 * i], chains[2 * i + 1])
                      for i in range(len(chains) // 2)]
        v = chains[0]
        top1 = jnp.max(v)
        mean8 = jnp.sum(jnp.where(lane >= L - K, v, jnp.float32(0.0))) * (1.0 / K)
        return jnp.where(lab == 0.0, top1, mean8)

    cp0.wait()
    vl0 = process_row(row0_v, r0)
    cp1.wait()
    vl1 = process_row(row1_v, r0 + 1)

    res = jnp.where(lane == 0, vl0, jnp.where(lane == 1, vl1, jnp.float32(0.0)))
    res_v[...] = res
    pltpu.sync_copy(res_v, out_hbm.at[wid])


_sc_topk = pl.kernel(
    _sc_topk_body,
    out_type=jax.ShapeDtypeStruct((NW, L), jnp.float32),
    mesh=plsc.VectorSubcoreMesh(core_axis_name="c", subcore_axis_name="s",
                                num_cores=NC, num_subcores=NS),
    compiler_params=pltpu.CompilerParams(needs_layout_passes=False),
    scratch_types=[
        pltpu.VMEM((N,), jnp.float32),
        pltpu.VMEM((N,), jnp.float32),
        pltpu.VMEM((B,), jnp.int32),
        pltpu.VMEM((B,), jnp.float32),
        pltpu.VMEM((L,), jnp.float32),
        pltpu.SemaphoreType.DMA,
        pltpu.SemaphoreType.DMA,
    ],
)


def _bce_body(vl_ref, lab_ref, out_ref):
    x = vl_ref[:, 0:ROWS_PER_W]
    y = lab_ref[...]
    t = jnp.maximum(x, 0.0) - x * y + jnp.log1p(jnp.exp(-jnp.abs(x)))
    out_ref[...] = jnp.mean(t).reshape(1, 1)


_bce = pl.pallas_call(
    _bce_body,
    out_shape=jax.ShapeDtypeStruct((1, 1), jnp.float32),
)


def kernel(scores, label, seqlen):
    packed = _sc_topk(scores, seqlen.astype(jnp.int32), label.astype(jnp.float32))
    lab2 = label.astype(jnp.float32).reshape(NW, ROWS_PER_W)
    return _bce(packed, lab2)[0, 0]
